# baseline (speedup 1.0000x reference)
import functools

import jax
import jax.numpy as jnp
from jax import lax
from jax.experimental import pallas as pl
from jax.experimental.pallas import tpu as pltpu
from jax.experimental.pallas import tpu_sc as plsc

_NC = 2
_NS = 16
_NW = _NC * _NS
_LANES = 16
_CH = 16
_UNROLL = 8


def kernel(x, pos_embedding):
    b, s, d = x.shape
    pos = pos_embedding.reshape(-1)
    x2 = x.reshape(b, s * d)
    rows_per_w = s // _NW
    n_chunks = rows_per_w // _CH
    ce = _CH * d
    n_steps = n_chunks * b
    mesh = plsc.VectorSubcoreMesh(core_axis_name="c", subcore_axis_name="s")

    @functools.partial(
        pl.kernel,
        mesh=mesh,
        out_type=jax.ShapeDtypeStruct((b, s * d), jnp.float32),
        scratch_types=[
            pltpu.VMEM((2, ce), jnp.float32),
            pltpu.VMEM((2, ce), jnp.float32),
            pltpu.SemaphoreType.DMA((2,)),
            pltpu.SemaphoreType.DMA((2,)),
            pltpu.SemaphoreType.DMA((2,)),
        ],
    )
    def k(x_hbm, pos_hbm, out_hbm, pos_v, x_v, sem_in, sem_out, sem_pos):
        wid = lax.axis_index("s") * _NC + lax.axis_index("c")
        base = wid * rows_per_w * d

        def chunk_off(c):
            return base + c * ce

        pltpu.async_copy(pos_hbm.at[pl.ds(chunk_off(0), ce)], pos_v.at[0],
                         sem_pos.at[0])
        pltpu.async_copy(x_hbm.at[0, pl.ds(chunk_off(0), ce)], x_v.at[0],
                         sem_in.at[0])

        def step(t, xb, pb):
            c = t // b
            bi = t % b
            off = chunk_off(c)
            if bi == 0 and c + 1 < n_chunks:
                pltpu.async_copy(
                    pos_hbm.at[pl.ds(chunk_off(c + 1), ce)],
                    pos_v.at[(pb + 1) % 2], sem_pos.at[(pb + 1) % 2])
            t1 = t + 1
            if t1 < n_steps:
                c1, b1 = t1 // b, t1 % b
                if t1 >= 2:
                    pltpu.make_async_copy(
                        x_v.at[(xb + 1) % 2],
                        out_hbm.at[(t1 - 2) % b, pl.ds(chunk_off((t1 - 2) // b), ce)],
                        sem_out.at[(xb + 1) % 2]).wait()
                pltpu.async_copy(
                    x_hbm.at[b1, pl.ds(chunk_off(c1), ce)],
                    x_v.at[(xb + 1) % 2], sem_in.at[(xb + 1) % 2])
            pltpu.make_async_copy(
                x_hbm.at[bi, pl.ds(off, ce)], x_v.at[xb], sem_in.at[xb]).wait()
            if bi == 0:
                pltpu.make_async_copy(
                    pos_hbm.at[pl.ds(off, ce)], pos_v.at[pb], sem_pos.at[pb]).wait()
            xbuf = x_v.at[xb]
            pbuf = pos_v.at[pb]

            def add_body(i, _):
                o0 = i * (_LANES * _UNROLL)
                for u in range(_UNROLL):
                    o = o0 + u * _LANES
                    xbuf[pl.ds(o, _LANES)] = (
                        xbuf[pl.ds(o, _LANES)] + pbuf[pl.ds(o, _LANES)])
                return 0

            lax.fori_loop(0, ce // (_LANES * _UNROLL), add_body, 0)
            pltpu.async_copy(xbuf, out_hbm.at[bi, pl.ds(off, ce)],
                             sem_out.at[xb])

        for t in range(n_steps):
            step(t, t % 2, (t // b) % 2)
        for t in (n_steps - 2, n_steps - 1):
            pltpu.make_async_copy(
                x_v.at[t % 2],
                out_hbm.at[t % b, pl.ds(chunk_off(t // b), ce)],
                sem_out.at[t % 2]).wait()

    out = k(x2, pos)
    return out.reshape(b, s, d)
